# Initial kernel scaffold; baseline (speedup 1.0000x reference)
import jax
import jax.numpy as jnp
from jax.experimental import pallas as pl


def kernel(input_ids, word_embeddings):
    raise NotImplementedError("write your pallas kernel here")


# speedup vs baseline: 1.4856x; 1.4856x over previous
import functools

import jax
import jax.numpy as jnp
from jax import lax
from jax.experimental import pallas as pl
from jax.experimental.pallas import tpu as pltpu
from jax.experimental.pallas import tpu_sc as plsc

_D = 4096
_B = 8192
_NC, _NS = 2, 16
_NW = _NC * _NS
_B_PER_W = _B // _NW
_R = 8
_NCHUNK = _B_PER_W // _R

_mesh = plsc.VectorSubcoreMesh(core_axis_name="c", subcore_axis_name="s")


@functools.partial(
    pl.kernel,
    mesh=_mesh,
    out_type=jax.ShapeDtypeStruct((_B, _D), jnp.float32),
    scratch_types=[
        pltpu.VMEM((_B_PER_W,), jnp.int32),
        pltpu.VMEM((_R, _D), jnp.float32),
        pltpu.SemaphoreType.DMA,
    ],
)
def _gather_kernel(ids_hbm, table_hbm, out_hbm, idx_v, rows_v, gsem):
    wid = lax.axis_index("s") * _NC + lax.axis_index("c")
    base = wid * _B_PER_W
    pltpu.sync_copy(ids_hbm.at[pl.ds(base, _B_PER_W)], idx_v)

    def chunk_body(c, carry):
        pltpu.async_copy(
            table_hbm.at[idx_v.at[pl.ds(c * _R, _R)]], rows_v, gsem
        ).wait()
        pltpu.sync_copy(rows_v, out_hbm.at[pl.ds(base + c * _R, _R)])
        return carry

    lax.fori_loop(0, _NCHUNK, chunk_body, 0)


def kernel(input_ids, word_embeddings):
    ids_flat = input_ids.reshape(-1).astype(jnp.int32)
    out = _gather_kernel(ids_flat, word_embeddings)
    return out.reshape(input_ids.shape + (word_embeddings.shape[1],))


# speedup vs baseline: 1.6826x; 1.1326x over previous
import functools

import jax
import jax.numpy as jnp
from jax import lax
from jax.experimental import pallas as pl
from jax.experimental.pallas import tpu as pltpu
from jax.experimental.pallas import tpu_sc as plsc

_D = 4096
_B = 8192
_NC, _NS = 2, 16
_NW = _NC * _NS
_B_PER_W = _B // _NW
_R = 8
_NCHUNK = _B_PER_W // _R
_NBUF = 2

_mesh = plsc.VectorSubcoreMesh(core_axis_name="c", subcore_axis_name="s")


@functools.partial(
    pl.kernel,
    mesh=_mesh,
    out_type=jax.ShapeDtypeStruct((_B, _D), jnp.float32),
    scratch_types=[
        pltpu.VMEM((_B_PER_W,), jnp.int32),
        pltpu.VMEM((_NBUF, _R, _D), jnp.float32),
    ]
    + [pltpu.SemaphoreType.DMA] * (2 * _NBUF),
)
def _gather_kernel(ids_hbm, table_hbm, out_hbm, idx_v, rows_v, *sems):
    gsems = sems[:_NBUF]
    ssems = sems[_NBUF:]
    wid = lax.axis_index("s") * _NC + lax.axis_index("c")
    base = wid * _B_PER_W
    pltpu.sync_copy(ids_hbm.at[pl.ds(base, _B_PER_W)], idx_v)

    def start_gather(chunk, b):
        pltpu.async_copy(
            table_hbm.at[idx_v.at[pl.ds(chunk * _R, _R)]], rows_v.at[b], gsems[b]
        )

    def gather_wait(b):
        pltpu.make_async_copy(
            table_hbm.at[pl.ds(0, _R)], rows_v.at[b], gsems[b]
        ).wait()

    def start_scatter(chunk, b):
        pltpu.async_copy(
            rows_v.at[b], out_hbm.at[pl.ds(base + chunk * _R, _R)], ssems[b]
        )

    def scatter_wait(b):
        pltpu.make_async_copy(
            rows_v.at[b], out_hbm.at[pl.ds(base, _R)], ssems[b]
        ).wait()

    for b in range(_NBUF):
        start_gather(b, b)

    def outer(r, carry):
        c = r * _NBUF
        for b in range(_NBUF):
            gather_wait(b)
            start_scatter(c + b, b)
        for b in range(_NBUF):
            nxt = c + _NBUF + b

            def refill(b=b, nxt=nxt):
                scatter_wait(b)
                start_gather(nxt, b)

            pl.when(nxt < _NCHUNK)(refill)
        return carry

    lax.fori_loop(0, _NCHUNK // _NBUF, outer, 0)

    for b in range(_NBUF):
        scatter_wait(b)


def kernel(input_ids, word_embeddings):
    ids_flat = input_ids.reshape(-1).astype(jnp.int32)
    out = _gather_kernel(ids_flat, word_embeddings)
    return out.reshape(input_ids.shape + (word_embeddings.shape[1],))


# speedup vs baseline: 1.6919x; 1.0055x over previous
import functools

import jax
import jax.numpy as jnp
from jax import lax
from jax.experimental import pallas as pl
from jax.experimental.pallas import tpu as pltpu
from jax.experimental.pallas import tpu_sc as plsc

_D = 4096
_B = 8192
_NC, _NS = 2, 16
_NW = _NC * _NS
_B_PER_W = _B // _NW
_R = 8
_NCHUNK = _B_PER_W // _R
_NBUF = 3
_NROUND = -(-_NCHUNK // _NBUF)

_mesh = plsc.VectorSubcoreMesh(core_axis_name="c", subcore_axis_name="s")


@functools.partial(
    pl.kernel,
    mesh=_mesh,
    out_type=jax.ShapeDtypeStruct((_B, _D), jnp.float32),
    scratch_types=[
        pltpu.VMEM((_B_PER_W,), jnp.int32),
        pltpu.VMEM((_NBUF, _R, _D), jnp.float32),
    ]
    + [pltpu.SemaphoreType.DMA] * (2 * _NBUF),
)
def _gather_kernel(ids_hbm, table_hbm, out_hbm, idx_v, rows_v, *sems):
    gsems = sems[:_NBUF]
    ssems = sems[_NBUF:]
    wid = lax.axis_index("s") * _NC + lax.axis_index("c")
    base = wid * _B_PER_W
    pltpu.sync_copy(ids_hbm.at[pl.ds(base, _B_PER_W)], idx_v)

    def start_gather(chunk, b):
        pltpu.async_copy(
            table_hbm.at[idx_v.at[pl.ds(chunk * _R, _R)]], rows_v.at[b], gsems[b]
        )

    def gather_wait(b):
        pltpu.make_async_copy(
            table_hbm.at[pl.ds(0, _R)], rows_v.at[b], gsems[b]
        ).wait()

    def start_scatter(chunk, b):
        pltpu.async_copy(
            rows_v.at[b], out_hbm.at[pl.ds(base + chunk * _R, _R)], ssems[b]
        )

    def scatter_wait(b):
        pltpu.make_async_copy(
            rows_v.at[b], out_hbm.at[pl.ds(base, _R)], ssems[b]
        ).wait()

    for b in range(_NBUF):
        start_gather(b, b)

    def outer(r, carry):
        c = r * _NBUF
        for b in range(_NBUF):
            chunk = c + b

            def drain(b=b, chunk=chunk):
                gather_wait(b)
                start_scatter(chunk, b)

            pl.when(chunk < _NCHUNK)(drain)
        for b in range(_NBUF):
            nxt = c + _NBUF + b

            def refill(b=b, nxt=nxt):
                scatter_wait(b)
                start_gather(nxt, b)

            pl.when(nxt < _NCHUNK)(refill)
        return carry

    lax.fori_loop(0, _NROUND, outer, 0)

    for b in range(_NBUF):
        scatter_wait(b)


def kernel(input_ids, word_embeddings):
    ids_flat = input_ids.reshape(-1).astype(jnp.int32)
    out = _gather_kernel(ids_flat, word_embeddings)
    return out.reshape(input_ids.shape + (word_embeddings.shape[1],))
